# baseline (speedup 1.0000x reference)
import functools

import jax
import jax.numpy as jnp
from jax import lax
from jax.experimental import pallas as pl
from jax.experimental.pallas import tpu as pltpu
from jax.experimental.pallas import tpu_sc as plsc

_B, _T, _D = 4, 2048, 768
_NC, _NS = 2, 16
_NW = _NC * _NS
_POS_PER_W = _T // _NW
_CHUNK = 32
_SUB = _POS_PER_W // _CHUNK
_NCHUNK = _B * _SUB
_LANES = 16
_HALF = _CHUNK // 2


def _make_embed_kernel():
    mesh = plsc.VectorSubcoreMesh(core_axis_name="c", subcore_axis_name="s")

    @functools.partial(
        pl.kernel,
        out_type=jax.ShapeDtypeStruct((_B, _T, _D), jnp.float32),
        mesh=mesh,
        scratch_types=[
            pltpu.VMEM((_POS_PER_W, _D), jnp.float32),
            pltpu.VMEM((_B, _POS_PER_W), jnp.int32),
            pltpu.VMEM((_CHUNK, _D), jnp.float32),
            pltpu.VMEM((_CHUNK, _D), jnp.float32),
            pltpu.SemaphoreType.DMA,
            pltpu.SemaphoreType.DMA,
            pltpu.SemaphoreType.DMA,
            pltpu.SemaphoreType.DMA,
            pltpu.SemaphoreType.DMA,
            pltpu.SemaphoreType.DMA,
            pltpu.SemaphoreType.DMA,
            pltpu.SemaphoreType.DMA,
            pltpu.SemaphoreType.DMA,
        ],
    )
    def embed(idx_hbm, tok_hbm, pos_hbm, out_hbm,
              pos_v, idx_v, rows0, rows1,
              sem_pos, sem_idx0, sem_idx,
              sem_g0lo, sem_g0hi, sem_g1lo, sem_g1hi, sem_o0, sem_o1):
        wid = lax.axis_index("s") * _NC + lax.axis_index("c")
        p0 = wid * _POS_PER_W
        rows = (rows0, rows1)
        sem_g = ((sem_g0lo, sem_g0hi), (sem_g1lo, sem_g1hi))
        sem_o = (sem_o0, sem_o1)

        pos_cp = pltpu.async_copy(pos_hbm.at[pl.ds(p0, _POS_PER_W)],
                                  pos_v, sem_pos)

        idx_cps = [
            pltpu.async_copy(idx_hbm.at[b, pl.ds(p0, _POS_PER_W)],
                             idx_v.at[b], sem_idx0 if b == 0 else sem_idx)
            for b in range(_B)
        ]

        def gather_half(b, s, buf, h):
            h0 = h * _HALF
            return pltpu.make_async_copy(
                tok_hbm.at[idx_v.at[b, pl.ds(s * _CHUNK + h0, _HALF)]],
                rows[buf].at[pl.ds(h0, _HALF)], sem_g[buf][h])

        def gather_start(b, s, buf):
            gather_half(b, s, buf, 0).start()
            gather_half(b, s, buf, 1).start()

        def out_half(b, buf, h0):
            return pltpu.make_async_copy(
                rows[buf].at[pl.ds(h0, _HALF)],
                out_hbm.at[b, pl.ds(p0 + buf * _CHUNK + h0, _HALF)],
                sem_o[buf])

        def drain_out(b, buf):
            pltpu.make_async_copy(
                rows[buf],
                out_hbm.at[b, pl.ds(p0 + buf * _CHUNK, _CHUNK)],
                sem_o[buf]).wait()

        def add_half(buf, h0):
            pos_base = buf * _CHUNK

            @plsc.parallel_loop(h0, h0 + _HALF, unroll=4)
            def _row(r):
                @plsc.parallel_loop(0, _D, step=_LANES, unroll=8)
                def _col(c):
                    plsc.addupdate(rows[buf].at[r, pl.ds(c, _LANES)],
                                   pos_v[pos_base + r, pl.ds(c, _LANES)])

        def add_and_out(b, s, buf, between=None):
            gather_half(b, s, buf, 0).wait()
            add_half(buf, 0)
            out_half(b, buf, 0).start()
            if between is not None:
                between()
            gather_half(b, s, buf, 1).wait()
            add_half(buf, _HALF)
            out_half(b, buf, _HALF).start()

        idx_cps[0].wait()
        gather_start(0, 0, 0)
        for cp in idx_cps[1:]:
            cp.wait()
        pos_cp.wait()

        @pl.loop(0, _B)
        def _batch(j):
            @pl.when(j > 0)
            def _():
                drain_out(j - 1, 1)
            gather_start(j, 1, 1)
            add_and_out(j, 0, 0)

            def _refill_buf0():
                @pl.when(j + 1 < _B)
                def _():
                    drain_out(j, 0)
                    gather_start(j + 1, 0, 0)

            add_and_out(j, 1, 1, between=_refill_buf0)

        drain_out(_B - 1, 0)
        drain_out(_B - 1, 1)

    return embed


_embed = _make_embed_kernel()


@jax.jit
def kernel(idx, tok_table, pos_table):
    return _embed(idx.astype(jnp.int32), tok_table, pos_table)


# speedup vs baseline: 1.0109x; 1.0109x over previous
import functools

import jax
import jax.numpy as jnp
from jax import lax
from jax.experimental import pallas as pl
from jax.experimental.pallas import tpu as pltpu
from jax.experimental.pallas import tpu_sc as plsc

_B, _T, _D = 4, 2048, 768
_NC, _NS = 2, 16
_NW = _NC * _NS
_POS_PER_W = _T // _NW
_CHUNK = 32
_SUB = _POS_PER_W // _CHUNK
_NCHUNK = _B * _SUB
_LANES = 16
_HALF = _CHUNK // 2


def _make_embed_kernel():
    mesh = plsc.VectorSubcoreMesh(core_axis_name="c", subcore_axis_name="s")

    @functools.partial(
        pl.kernel,
        out_type=jax.ShapeDtypeStruct((_B, _T, _D), jnp.float32),
        mesh=mesh,
        scratch_types=[
            pltpu.VMEM((_POS_PER_W, _D), jnp.float32),
            pltpu.VMEM((_B, _POS_PER_W), jnp.int32),
            pltpu.VMEM((_CHUNK, _D), jnp.float32),
            pltpu.VMEM((_CHUNK, _D), jnp.float32),
            pltpu.SemaphoreType.DMA,
            pltpu.SemaphoreType.DMA,
            pltpu.SemaphoreType.DMA,
            pltpu.SemaphoreType.DMA,
            pltpu.SemaphoreType.DMA,
            pltpu.SemaphoreType.DMA,
            pltpu.SemaphoreType.DMA,
            pltpu.SemaphoreType.DMA,
            pltpu.SemaphoreType.DMA,
        ],
    )
    def embed(idx_hbm, tok_hbm, pos_hbm, out_hbm,
              pos_v, idx_v, rows0, rows1,
              sem_pos, sem_idx0, sem_idx,
              sem_g0lo, sem_g0hi, sem_g1lo, sem_g1hi, sem_o0, sem_o1):
        wid = lax.axis_index("s") * _NC + lax.axis_index("c")
        p0 = wid * _POS_PER_W
        rows = (rows0, rows1)
        sem_g = ((sem_g0lo, sem_g0hi), (sem_g1lo, sem_g1hi))
        sem_o = (sem_o0, sem_o1)

        pos_cp = pltpu.async_copy(pos_hbm.at[pl.ds(p0, _POS_PER_W)],
                                  pos_v, sem_pos)

        idx_cps = [
            pltpu.async_copy(idx_hbm.at[b, pl.ds(p0, _POS_PER_W)],
                             idx_v.at[b], sem_idx0 if b == 0 else sem_idx)
            for b in range(_B)
        ]

        def gather_half(b, s, buf, h):
            h0 = h * _HALF
            return pltpu.make_async_copy(
                tok_hbm.at[idx_v.at[b, pl.ds(s * _CHUNK + h0, _HALF)]],
                rows[buf].at[pl.ds(h0, _HALF)], sem_g[buf][h])

        def gather_start(b, s, buf):
            gather_half(b, s, buf, 0).start()
            gather_half(b, s, buf, 1).start()

        def out_half(b, buf, h0):
            return pltpu.make_async_copy(
                rows[buf].at[pl.ds(h0, _HALF)],
                out_hbm.at[b, pl.ds(p0 + buf * _CHUNK + h0, _HALF)],
                sem_o[buf])

        def drain_out(b, buf):
            pltpu.make_async_copy(
                rows[buf],
                out_hbm.at[b, pl.ds(p0 + buf * _CHUNK, _CHUNK)],
                sem_o[buf]).wait()

        def add_half(buf, h0):
            pos_base = buf * _CHUNK

            @plsc.parallel_loop(h0, h0 + _HALF, unroll=2)
            def _row(r):
                @plsc.parallel_loop(0, _D, step=_LANES, unroll=8)
                def _col(c):
                    plsc.addupdate(rows[buf].at[r, pl.ds(c, _LANES)],
                                   pos_v[pos_base + r, pl.ds(c, _LANES)])

        def add_and_out(b, s, buf, between=None):
            gather_half(b, s, buf, 0).wait()
            add_half(buf, 0)
            out_half(b, buf, 0).start()
            if between is not None:
                between()
            gather_half(b, s, buf, 1).wait()
            add_half(buf, _HALF)
            out_half(b, buf, _HALF).start()

        idx_cps[0].wait()
        gather_start(0, 0, 0)
        for cp in idx_cps[1:]:
            cp.wait()
        pos_cp.wait()

        @pl.loop(0, _B)
        def _batch(j):
            @pl.when(j > 0)
            def _():
                drain_out(j - 1, 1)
            gather_start(j, 1, 1)
            add_and_out(j, 0, 0)

            def _refill_buf0():
                @pl.when(j + 1 < _B)
                def _():
                    drain_out(j, 0)
                    gather_start(j + 1, 0, 0)

            add_and_out(j, 1, 1, between=_refill_buf0)

        drain_out(_B - 1, 0)
        drain_out(_B - 1, 1)

    return embed


_embed = _make_embed_kernel()


@jax.jit
def kernel(idx, tok_table, pos_table):
    return _embed(idx.astype(jnp.int32), tok_table, pos_table)
